# baseline (speedup 1.0000x reference)
import functools

import jax
import jax.numpy as jnp
from jax import lax
from jax.experimental import pallas as pl
from jax.experimental.pallas import tpu as pltpu
from jax.experimental.pallas import tpu_sc as plsc

_K = 512
_NC = 2
_NS = 16
_NW = _NC * _NS
_L = 16


def _vq_body(n_chunk, w_hbm, rank_hbm, x_hbm, out_hbm, part_hbm,
             w_v, rank_v, codes_v, mids_v, x_v, out_v, acc_v):
    wid = lax.axis_index("s") * _NC + lax.axis_index("c")
    base = wid * n_chunk
    pltpu.sync_copy(w_hbm, w_v)
    pltpu.sync_copy(rank_hbm, rank_v)
    pltpu.sync_copy(x_hbm.at[pl.ds(base, n_chunk)], x_v)

    lane = lax.iota(jnp.int32, _L)

    def place(c, carry):
        wv = w_v[pl.ds(c * _L, _L)]
        r16 = rank_v[pl.ds(c * _L, _L)] * _L
        for m in range(_L):
            slot = (lane + m) & (_L - 1)
            plsc.store_scatter(codes_v, [r16 + slot], wv)
        return carry

    plsc.parallel_loop(0, _K // _L, carry=jnp.int32(0))(place)

    def mid(c, carry):
        a = codes_v[pl.ds(c * _L, _L)]
        b = codes_v[pl.ds(c * _L + _L, _L)]
        mids_v[pl.ds(c * _L, _L)] = (a + b) * 0.5
        return carry

    plsc.parallel_loop(0, _K - 1, carry=jnp.int32(0))(mid)
    mids_v[pl.ds((_K - 1) * _L, _L)] = jnp.full((_L,), jnp.inf, jnp.float32)

    pivots = [
        plsc.load_gather(mids_v, [lane + _L * (64 * k - 1)])
        for k in range(1, 8)
    ]

    def body(i, acc):
        x = x_v[pl.ds(i * _L, _L)]
        bits = [(p <= x).astype(jnp.int32) for p in pivots]
        while len(bits) > 1:
            bits = [a + b for a, b in zip(bits[::2], bits[1::2])] + (
                [bits[-1]] if len(bits) % 2 else [])
        j16 = bits[0] * (64 * _L) + lane
        for w in (32, 16, 8, 4, 2, 1):
            m = plsc.load_gather(mids_v, [j16 + _L * (w - 1)])
            j16 = jnp.where(m <= x, j16 + _L * w, j16)
        q = plsc.load_gather(codes_v, [j16])
        d = x - q
        out_v[pl.ds(i * _L, _L)] = x + (q - x)
        return acc + d * d

    acc = plsc.parallel_loop(
        0, n_chunk // _L, unroll=8,
        carry=jnp.zeros((_L,), jnp.float32))(body)
    acc_v[...] = acc
    pltpu.sync_copy(out_v, out_hbm.at[pl.ds(base, n_chunk)])
    pltpu.sync_copy(acc_v, part_hbm.at[pl.ds(wid * _L, _L)])


def kernel(input, weight):
    shape = input.shape
    x = input.reshape(-1)
    n = x.size
    n_chunk = n // _NW
    w = weight.reshape(-1)
    ar = jnp.arange(_K)
    lt = w[None, :] < w[:, None]
    tie = (w[None, :] == w[:, None]) & (ar[None, :] < ar[:, None])
    rank = jnp.sum(lt | tie, axis=1).astype(jnp.int32)
    mesh = plsc.VectorSubcoreMesh(core_axis_name="c", subcore_axis_name="s")
    out, part = pl.kernel(
        functools.partial(_vq_body, n_chunk),
        out_type=(jax.ShapeDtypeStruct((n,), jnp.float32),
                  jax.ShapeDtypeStruct((_NW * _L,), jnp.float32)),
        mesh=mesh,
        compiler_params=pltpu.CompilerParams(needs_layout_passes=False),
        scratch_types=[
            pltpu.VMEM((_K,), jnp.float32),
            pltpu.VMEM((_K,), jnp.int32),
            pltpu.VMEM((_K * _L,), jnp.float32),
            pltpu.VMEM((_K * _L,), jnp.float32),
            pltpu.VMEM((n_chunk,), jnp.float32),
            pltpu.VMEM((n_chunk,), jnp.float32),
            pltpu.VMEM((_L,), jnp.float32),
        ],
    )(w, rank, x)
    loss = jnp.sum(part) / n
    return out.reshape(shape), loss, loss


# speedup vs baseline: 1.0716x; 1.0716x over previous
import functools

import jax
import jax.numpy as jnp
from jax import lax
from jax.experimental import pallas as pl
from jax.experimental.pallas import tpu as pltpu
from jax.experimental.pallas import tpu_sc as plsc

_K = 512
_NC = 1
_NS = 16
_NW = _NC * _NS
_L = 16


def _vq_body(n_chunk, w_hbm, rank_hbm, x_hbm, out_hbm, part_hbm,
             w_v, rank_v, codes_v, mids_v, x_v, out_v, acc_v):
    wid = lax.axis_index("s") * _NC + lax.axis_index("c")
    base = wid * n_chunk
    pltpu.sync_copy(w_hbm, w_v)
    pltpu.sync_copy(rank_hbm, rank_v)
    pltpu.sync_copy(x_hbm.at[pl.ds(base, n_chunk)], x_v)

    lane = lax.iota(jnp.int32, _L)

    def place(c, carry):
        wv = w_v[pl.ds(c * _L, _L)]
        r16 = rank_v[pl.ds(c * _L, _L)] * _L
        for m in range(_L):
            slot = (lane + m) & (_L - 1)
            plsc.store_scatter(codes_v, [r16 + slot], wv)
        return carry

    plsc.parallel_loop(0, _K // _L, carry=jnp.int32(0))(place)

    def mid(c, carry):
        a = codes_v[pl.ds(c * _L, _L)]
        b = codes_v[pl.ds(c * _L + _L, _L)]
        mids_v[pl.ds(c * _L, _L)] = (a + b) * 0.5
        return carry

    plsc.parallel_loop(0, _K - 1, carry=jnp.int32(0))(mid)
    mids_v[pl.ds((_K - 1) * _L, _L)] = jnp.full((_L,), jnp.inf, jnp.float32)

    pivots = [
        plsc.load_gather(mids_v, [lane + _L * (64 * k - 1)])
        for k in range(1, 8)
    ]

    def body(i, acc):
        x = x_v[pl.ds(i * _L, _L)]
        bits = [(p <= x).astype(jnp.int32) for p in pivots]
        while len(bits) > 1:
            bits = [a + b for a, b in zip(bits[::2], bits[1::2])] + (
                [bits[-1]] if len(bits) % 2 else [])
        j16 = bits[0] * (64 * _L) + lane
        for w in (32, 16, 8, 4, 2, 1):
            m = plsc.load_gather(mids_v, [j16 + _L * (w - 1)])
            j16 = jnp.where(m <= x, j16 + _L * w, j16)
        q = plsc.load_gather(codes_v, [j16])
        d = x - q
        out_v[pl.ds(i * _L, _L)] = x + (q - x)
        return acc + d * d

    acc = plsc.parallel_loop(
        0, n_chunk // _L, unroll=4,
        carry=jnp.zeros((_L,), jnp.float32))(body)
    acc_v[...] = acc
    pltpu.sync_copy(out_v, out_hbm.at[pl.ds(base, n_chunk)])
    pltpu.sync_copy(acc_v, part_hbm.at[pl.ds(wid * _L, _L)])


def kernel(input, weight):
    shape = input.shape
    x = input.reshape(-1)
    n = x.size
    n_chunk = n // _NW
    w = weight.reshape(-1)
    ar = jnp.arange(_K)
    lt = w[None, :] < w[:, None]
    tie = (w[None, :] == w[:, None]) & (ar[None, :] < ar[:, None])
    rank = jnp.sum(lt | tie, axis=1).astype(jnp.int32)
    mesh = plsc.VectorSubcoreMesh(core_axis_name="c", subcore_axis_name="s", num_cores=1)
    out, part = pl.kernel(
        functools.partial(_vq_body, n_chunk),
        out_type=(jax.ShapeDtypeStruct((n,), jnp.float32),
                  jax.ShapeDtypeStruct((_NW * _L,), jnp.float32)),
        mesh=mesh,
        compiler_params=pltpu.CompilerParams(needs_layout_passes=False),
        scratch_types=[
            pltpu.VMEM((_K,), jnp.float32),
            pltpu.VMEM((_K,), jnp.int32),
            pltpu.VMEM((_K * _L,), jnp.float32),
            pltpu.VMEM((_K * _L,), jnp.float32),
            pltpu.VMEM((n_chunk,), jnp.float32),
            pltpu.VMEM((n_chunk,), jnp.float32),
            pltpu.VMEM((_L,), jnp.float32),
        ],
    )(w, rank, x)
    loss = jnp.sum(part) / n
    return out.reshape(shape), loss, loss


# speedup vs baseline: 1.1258x; 1.0505x over previous
import functools

import jax
import jax.numpy as jnp
from jax import lax
from jax.experimental import pallas as pl
from jax.experimental.pallas import tpu as pltpu
from jax.experimental.pallas import tpu_sc as plsc

_K = 512
_NC = 2
_NS = 16
_NW = _NC * _NS
_L = 16


def _vq_body(n_chunk, w_hbm, rank_hbm, x_hbm, out_hbm, part_hbm,
             w_v, rank_v, codes_v, mids_v, x_v, out_v, acc_v):
    wid = lax.axis_index("s") * _NC + lax.axis_index("c")
    base = wid * n_chunk
    pltpu.sync_copy(w_hbm, w_v)
    pltpu.sync_copy(rank_hbm, rank_v)
    pltpu.sync_copy(x_hbm.at[pl.ds(base, n_chunk)], x_v)

    lane = lax.iota(jnp.int32, _L)

    def place(c, carry):
        wv = w_v[pl.ds(c * _L, _L)]
        r16 = rank_v[pl.ds(c * _L, _L)] * _L
        for m in range(_L):
            slot = (lane + m) & (_L - 1)
            plsc.store_scatter(codes_v, [r16 + slot], wv)
        return carry

    plsc.parallel_loop(0, _K // _L, carry=jnp.int32(0))(place)

    def mid(c, carry):
        a = codes_v[pl.ds(c * _L, _L)]
        b = codes_v[pl.ds(c * _L + _L, _L)]
        mids_v[pl.ds(c * _L, _L)] = (a + b) * 0.5
        return carry

    plsc.parallel_loop(0, _K - 1, carry=jnp.int32(0))(mid)
    mids_v[pl.ds((_K - 1) * _L, _L)] = jnp.full((_L,), jnp.inf, jnp.float32)

    pivots = [
        plsc.load_gather(mids_v, [lane + _L * (64 * k - 1)])
        for k in range(1, 8)
    ]

    def body(i, acc):
        x = x_v[pl.ds(i * _L, _L)]
        bits = [(p <= x).astype(jnp.int32) for p in pivots]
        while len(bits) > 1:
            bits = [a + b for a, b in zip(bits[::2], bits[1::2])] + (
                [bits[-1]] if len(bits) % 2 else [])
        j16 = bits[0] * (64 * _L) + lane
        for w in (32, 16, 8, 4, 2, 1):
            m = plsc.load_gather(mids_v, [j16 + _L * (w - 1)])
            j16 = jnp.where(m <= x, j16 + _L * w, j16)
        q = plsc.load_gather(codes_v, [j16])
        d = x - q
        out_v[pl.ds(i * _L, _L)] = x + (q - x)
        return acc + d * d

    acc = plsc.parallel_loop(
        0, n_chunk // _L, unroll=6,
        carry=jnp.zeros((_L,), jnp.float32))(body)
    acc_v[...] = acc
    pltpu.sync_copy(out_v, out_hbm.at[pl.ds(base, n_chunk)])
    pltpu.sync_copy(acc_v, part_hbm.at[pl.ds(wid * _L, _L)])


def kernel(input, weight):
    shape = input.shape
    x = input.reshape(-1)
    n = x.size
    n_chunk = n // _NW
    w = weight.reshape(-1)
    ar = jnp.arange(_K)
    lt = w[None, :] < w[:, None]
    tie = (w[None, :] == w[:, None]) & (ar[None, :] < ar[:, None])
    rank = jnp.sum(lt | tie, axis=1).astype(jnp.int32)
    mesh = plsc.VectorSubcoreMesh(core_axis_name="c", subcore_axis_name="s")
    out, part = pl.kernel(
        functools.partial(_vq_body, n_chunk),
        out_type=(jax.ShapeDtypeStruct((n,), jnp.float32),
                  jax.ShapeDtypeStruct((_NW * _L,), jnp.float32)),
        mesh=mesh,
        compiler_params=pltpu.CompilerParams(needs_layout_passes=False),
        scratch_types=[
            pltpu.VMEM((_K,), jnp.float32),
            pltpu.VMEM((_K,), jnp.int32),
            pltpu.VMEM((_K * _L,), jnp.float32),
            pltpu.VMEM((_K * _L,), jnp.float32),
            pltpu.VMEM((n_chunk,), jnp.float32),
            pltpu.VMEM((n_chunk,), jnp.float32),
            pltpu.VMEM((_L,), jnp.float32),
        ],
    )(w, rank, x)
    loss = jnp.sum(part) / n
    return out.reshape(shape), loss, loss


# speedup vs baseline: 1.1267x; 1.0008x over previous
import functools

import jax
import jax.numpy as jnp
from jax import lax
from jax.experimental import pallas as pl
from jax.experimental.pallas import tpu as pltpu
from jax.experimental.pallas import tpu_sc as plsc

_K = 512
_NC = 2
_NS = 16
_NW = _NC * _NS
_L = 16


def _vq_body(n_chunk, w_hbm, rank_hbm, x_hbm, out_hbm, part_hbm,
             w_v, rank_v, codes_v, mids_v, x_v, out_v, acc_v):
    wid = lax.axis_index("s") * _NC + lax.axis_index("c")
    base = wid * n_chunk
    pltpu.sync_copy(w_hbm, w_v)
    pltpu.sync_copy(rank_hbm, rank_v)
    pltpu.sync_copy(x_hbm.at[pl.ds(base, n_chunk)], x_v)

    lane = lax.iota(jnp.int32, _L)

    def place(c, carry):
        wv = w_v[pl.ds(c * _L, _L)]
        r16 = rank_v[pl.ds(c * _L, _L)] * _L
        for m in range(_L):
            slot = (lane + m) & (_L - 1)
            plsc.store_scatter(codes_v, [r16 + slot], wv)
        return carry

    plsc.parallel_loop(0, _K // _L, carry=jnp.int32(0))(place)

    def mid(c, carry):
        a = codes_v[pl.ds(c * _L, _L)]
        b = codes_v[pl.ds(c * _L + _L, _L)]
        mids_v[pl.ds(c * _L, _L)] = (a + b) * 0.5
        return carry

    plsc.parallel_loop(0, _K - 1, carry=jnp.int32(0))(mid)
    mids_v[pl.ds((_K - 1) * _L, _L)] = jnp.full((_L,), jnp.inf, jnp.float32)

    pivots = [
        plsc.load_gather(mids_v, [lane + _L * (64 * k - 1)])
        for k in range(1, 8)
    ]

    def body(i, acc):
        x = x_v[pl.ds(i * _L, _L)]
        bits = [(p <= x).astype(jnp.int32) for p in pivots]
        while len(bits) > 1:
            bits = [a + b for a, b in zip(bits[::2], bits[1::2])] + (
                [bits[-1]] if len(bits) % 2 else [])
        j16 = bits[0] * (64 * _L) + lane
        for w in (32, 16, 8, 4, 2, 1):
            m = plsc.load_gather(mids_v, [j16 + _L * (w - 1)])
            j16 = jnp.where(m <= x, j16 + _L * w, j16)
        q = plsc.load_gather(codes_v, [j16])
        d = x - q
        out_v[pl.ds(i * _L, _L)] = x + (q - x)
        return acc + d * d

    acc = plsc.parallel_loop(
        0, n_chunk // _L, unroll=7,
        carry=jnp.zeros((_L,), jnp.float32))(body)
    acc_v[...] = acc
    pltpu.sync_copy(out_v, out_hbm.at[pl.ds(base, n_chunk)])
    pltpu.sync_copy(acc_v, part_hbm.at[pl.ds(wid * _L, _L)])


def kernel(input, weight):
    shape = input.shape
    x = input.reshape(-1)
    n = x.size
    n_chunk = n // _NW
    w = weight.reshape(-1)
    ar = jnp.arange(_K)
    lt = w[None, :] < w[:, None]
    tie = (w[None, :] == w[:, None]) & (ar[None, :] < ar[:, None])
    rank = jnp.sum(lt | tie, axis=1).astype(jnp.int32)
    mesh = plsc.VectorSubcoreMesh(core_axis_name="c", subcore_axis_name="s")
    out, part = pl.kernel(
        functools.partial(_vq_body, n_chunk),
        out_type=(jax.ShapeDtypeStruct((n,), jnp.float32),
                  jax.ShapeDtypeStruct((_NW * _L,), jnp.float32)),
        mesh=mesh,
        compiler_params=pltpu.CompilerParams(needs_layout_passes=False),
        scratch_types=[
            pltpu.VMEM((_K,), jnp.float32),
            pltpu.VMEM((_K,), jnp.int32),
            pltpu.VMEM((_K * _L,), jnp.float32),
            pltpu.VMEM((_K * _L,), jnp.float32),
            pltpu.VMEM((n_chunk,), jnp.float32),
            pltpu.VMEM((n_chunk,), jnp.float32),
            pltpu.VMEM((_L,), jnp.float32),
        ],
    )(w, rank, x)
    loss = jnp.sum(part) / n
    return out.reshape(shape), loss, loss


# speedup vs baseline: 1.1810x; 1.0482x over previous
import functools

import jax
import jax.numpy as jnp
from jax import lax
from jax.experimental import pallas as pl
from jax.experimental.pallas import tpu as pltpu
from jax.experimental.pallas import tpu_sc as plsc

_K = 512
_NC = 2
_NS = 16
_NW = _NC * _NS
_L = 16


def _vq_body(n_chunk, w_hbm, rank_hbm, x_hbm, out_hbm, part_hbm,
             w_v, rank_v, codes_v, mids_v, x_v, out_v, acc_v):
    wid = lax.axis_index("s") * _NC + lax.axis_index("c")
    base = wid * n_chunk
    pltpu.sync_copy(w_hbm, w_v)
    pltpu.sync_copy(rank_hbm, rank_v)
    pltpu.sync_copy(x_hbm.at[pl.ds(base, n_chunk)], x_v)

    lane = lax.iota(jnp.int32, _L)

    def place(c, carry):
        wv = w_v[pl.ds(c * _L, _L)]
        r16 = rank_v[pl.ds(c * _L, _L)] * _L
        for m in range(_L):
            slot = (lane + m) & (_L - 1)
            plsc.store_scatter(codes_v, [r16 + slot], wv)
        return carry

    plsc.parallel_loop(0, _K // _L, carry=jnp.int32(0))(place)

    def mid(c, carry):
        a = codes_v[pl.ds(c * _L, _L)]
        b = codes_v[pl.ds(c * _L + _L, _L)]
        mids_v[pl.ds(c * _L, _L)] = (a + b) * 0.5
        return carry

    plsc.parallel_loop(0, _K - 1, carry=jnp.int32(0))(mid)
    mids_v[pl.ds((_K - 1) * _L, _L)] = jnp.full((_L,), jnp.inf, jnp.float32)

    pivots = [
        plsc.load_gather(mids_v, [lane + _L * (128 * k - 1)])
        for k in range(1, 4)
    ]

    def body(i, acc):
        x = x_v[pl.ds(i * _L, _L)]
        bits = [(p <= x).astype(jnp.int32) for p in pivots]
        while len(bits) > 1:
            bits = [a + b for a, b in zip(bits[::2], bits[1::2])] + (
                [bits[-1]] if len(bits) % 2 else [])
        j16 = bits[0] * (128 * _L) + lane
        for w in (64, 32, 16, 8, 4, 2, 1):
            m = plsc.load_gather(mids_v, [j16 + _L * (w - 1)])
            j16 = jnp.where(m <= x, j16 + _L * w, j16)
        q = plsc.load_gather(codes_v, [j16])
        d = x - q
        out_v[pl.ds(i * _L, _L)] = x + (q - x)
        return acc + d * d

    acc = plsc.parallel_loop(
        0, n_chunk // _L, unroll=7,
        carry=jnp.zeros((_L,), jnp.float32))(body)
    acc_v[...] = acc
    pltpu.sync_copy(out_v, out_hbm.at[pl.ds(base, n_chunk)])
    pltpu.sync_copy(acc_v, part_hbm.at[pl.ds(wid * _L, _L)])


def kernel(input, weight):
    shape = input.shape
    x = input.reshape(-1)
    n = x.size
    n_chunk = n // _NW
    w = weight.reshape(-1)
    ar = jnp.arange(_K)
    lt = w[None, :] < w[:, None]
    tie = (w[None, :] == w[:, None]) & (ar[None, :] < ar[:, None])
    rank = jnp.sum(lt | tie, axis=1).astype(jnp.int32)
    mesh = plsc.VectorSubcoreMesh(core_axis_name="c", subcore_axis_name="s")
    out, part = pl.kernel(
        functools.partial(_vq_body, n_chunk),
        out_type=(jax.ShapeDtypeStruct((n,), jnp.float32),
                  jax.ShapeDtypeStruct((_NW * _L,), jnp.float32)),
        mesh=mesh,
        compiler_params=pltpu.CompilerParams(needs_layout_passes=False),
        scratch_types=[
            pltpu.VMEM((_K,), jnp.float32),
            pltpu.VMEM((_K,), jnp.int32),
            pltpu.VMEM((_K * _L,), jnp.float32),
            pltpu.VMEM((_K * _L,), jnp.float32),
            pltpu.VMEM((n_chunk,), jnp.float32),
            pltpu.VMEM((n_chunk,), jnp.float32),
            pltpu.VMEM((_L,), jnp.float32),
        ],
    )(w, rank, x)
    loss = jnp.sum(part) / n
    return out.reshape(shape), loss, loss


# speedup vs baseline: 1.1892x; 1.0070x over previous
import functools

import jax
import jax.numpy as jnp
from jax import lax
from jax.experimental import pallas as pl
from jax.experimental.pallas import tpu as pltpu
from jax.experimental.pallas import tpu_sc as plsc

_K = 512
_NC = 2
_NS = 16
_NW = _NC * _NS
_L = 16


def _vq_body(n_chunk, w_hbm, rank_hbm, x_hbm, out_hbm, part_hbm,
             w_v, rank_v, codes_v, mids_v, x_v, out_v, acc_v):
    wid = lax.axis_index("s") * _NC + lax.axis_index("c")
    base = wid * n_chunk
    pltpu.sync_copy(w_hbm, w_v)
    pltpu.sync_copy(rank_hbm, rank_v)
    pltpu.sync_copy(x_hbm.at[pl.ds(base, n_chunk)], x_v)

    lane = lax.iota(jnp.int32, _L)

    def place(c, carry):
        wv = w_v[pl.ds(c * _L, _L)]
        r16 = rank_v[pl.ds(c * _L, _L)] * _L
        for m in range(_L):
            slot = (lane + m) & (_L - 1)
            plsc.store_scatter(codes_v, [r16 + slot], wv)
        return carry

    plsc.parallel_loop(0, _K // _L, carry=jnp.int32(0))(place)

    def mid(c, carry):
        a = codes_v[pl.ds(c * _L, _L)]
        b = codes_v[pl.ds(c * _L + _L, _L)]
        mids_v[pl.ds(c * _L, _L)] = (a + b) * 0.5
        return carry

    plsc.parallel_loop(0, _K - 1, carry=jnp.int32(0))(mid)
    mids_v[pl.ds((_K - 1) * _L, _L)] = jnp.full((_L,), jnp.inf, jnp.float32)

    pivots = [
        plsc.load_gather(mids_v, [lane + _L * (256 * k - 1)])
        for k in range(1, 2)
    ]

    def body(i, acc):
        x = x_v[pl.ds(i * _L, _L)]
        bits = [(p <= x).astype(jnp.int32) for p in pivots]
        while len(bits) > 1:
            bits = [a + b for a, b in zip(bits[::2], bits[1::2])] + (
                [bits[-1]] if len(bits) % 2 else [])
        j16 = bits[0] * (256 * _L) + lane
        for w in (128, 64, 32, 16, 8, 4, 2, 1):
            m = plsc.load_gather(mids_v, [j16 + _L * (w - 1)])
            j16 = jnp.where(m <= x, j16 + _L * w, j16)
        q = plsc.load_gather(codes_v, [j16])
        d = x - q
        out_v[pl.ds(i * _L, _L)] = x + (q - x)
        return acc + d * d

    acc = plsc.parallel_loop(
        0, n_chunk // _L, unroll=7,
        carry=jnp.zeros((_L,), jnp.float32))(body)
    acc_v[...] = acc
    pltpu.sync_copy(out_v, out_hbm.at[pl.ds(base, n_chunk)])
    pltpu.sync_copy(acc_v, part_hbm.at[pl.ds(wid * _L, _L)])


def kernel(input, weight):
    shape = input.shape
    x = input.reshape(-1)
    n = x.size
    n_chunk = n // _NW
    w = weight.reshape(-1)
    ar = jnp.arange(_K)
    lt = w[None, :] < w[:, None]
    tie = (w[None, :] == w[:, None]) & (ar[None, :] < ar[:, None])
    rank = jnp.sum(lt | tie, axis=1).astype(jnp.int32)
    mesh = plsc.VectorSubcoreMesh(core_axis_name="c", subcore_axis_name="s")
    out, part = pl.kernel(
        functools.partial(_vq_body, n_chunk),
        out_type=(jax.ShapeDtypeStruct((n,), jnp.float32),
                  jax.ShapeDtypeStruct((_NW * _L,), jnp.float32)),
        mesh=mesh,
        compiler_params=pltpu.CompilerParams(needs_layout_passes=False),
        scratch_types=[
            pltpu.VMEM((_K,), jnp.float32),
            pltpu.VMEM((_K,), jnp.int32),
            pltpu.VMEM((_K * _L,), jnp.float32),
            pltpu.VMEM((_K * _L,), jnp.float32),
            pltpu.VMEM((n_chunk,), jnp.float32),
            pltpu.VMEM((n_chunk,), jnp.float32),
            pltpu.VMEM((_L,), jnp.float32),
        ],
    )(w, rank, x)
    loss = jnp.sum(part) / n
    return out.reshape(shape), loss, loss


# speedup vs baseline: 1.1931x; 1.0033x over previous
import functools

import jax
import jax.numpy as jnp
from jax import lax
from jax.experimental import pallas as pl
from jax.experimental.pallas import tpu as pltpu
from jax.experimental.pallas import tpu_sc as plsc

_K = 512
_NC = 2
_NS = 16
_NW = _NC * _NS
_L = 16


def _vq_body(n_chunk, w_hbm, rank_hbm, x_hbm, out_hbm, part_hbm,
             w_v, rank_v, codes_v, mids_v, x_v, out_v, acc_v):
    wid = lax.axis_index("s") * _NC + lax.axis_index("c")
    base = wid * n_chunk
    pltpu.sync_copy(w_hbm, w_v)
    pltpu.sync_copy(rank_hbm, rank_v)
    pltpu.sync_copy(x_hbm.at[pl.ds(base, n_chunk)], x_v)

    lane = lax.iota(jnp.int32, _L)

    def place(c, carry):
        wv = w_v[pl.ds(c * _L, _L)]
        r16 = rank_v[pl.ds(c * _L, _L)] * _L
        for m in range(_L):
            slot = (lane + m) & (_L - 1)
            plsc.store_scatter(codes_v, [r16 + slot], wv)
        return carry

    plsc.parallel_loop(0, _K // _L, carry=jnp.int32(0))(place)

    def mid(c, carry):
        a = codes_v[pl.ds(c * _L, _L)]
        b = codes_v[pl.ds(c * _L + _L, _L)]
        mids_v[pl.ds(c * _L, _L)] = (a + b) * 0.5
        return carry

    plsc.parallel_loop(0, _K - 1, carry=jnp.int32(0))(mid)
    mids_v[pl.ds((_K - 1) * _L, _L)] = jnp.full((_L,), jnp.inf, jnp.float32)


    def body(i, acc):
        x = x_v[pl.ds(i * _L, _L)]
        j16 = lane
        for w in (256, 128, 64, 32, 16, 8, 4, 2, 1):
            m = plsc.load_gather(mids_v, [j16 + _L * (w - 1)])
            j16 = jnp.where(m <= x, j16 + _L * w, j16)
        q = plsc.load_gather(codes_v, [j16])
        d = x - q
        out_v[pl.ds(i * _L, _L)] = x + (q - x)
        return acc + d * d

    acc = plsc.parallel_loop(
        0, n_chunk // _L, unroll=7,
        carry=jnp.zeros((_L,), jnp.float32))(body)
    acc_v[...] = acc
    pltpu.sync_copy(out_v, out_hbm.at[pl.ds(base, n_chunk)])
    pltpu.sync_copy(acc_v, part_hbm.at[pl.ds(wid * _L, _L)])


def kernel(input, weight):
    shape = input.shape
    x = input.reshape(-1)
    n = x.size
    n_chunk = n // _NW
    w = weight.reshape(-1)
    ar = jnp.arange(_K)
    lt = w[None, :] < w[:, None]
    tie = (w[None, :] == w[:, None]) & (ar[None, :] < ar[:, None])
    rank = jnp.sum(lt | tie, axis=1).astype(jnp.int32)
    mesh = plsc.VectorSubcoreMesh(core_axis_name="c", subcore_axis_name="s")
    out, part = pl.kernel(
        functools.partial(_vq_body, n_chunk),
        out_type=(jax.ShapeDtypeStruct((n,), jnp.float32),
                  jax.ShapeDtypeStruct((_NW * _L,), jnp.float32)),
        mesh=mesh,
        compiler_params=pltpu.CompilerParams(needs_layout_passes=False),
        scratch_types=[
            pltpu.VMEM((_K,), jnp.float32),
            pltpu.VMEM((_K,), jnp.int32),
            pltpu.VMEM((_K * _L,), jnp.float32),
            pltpu.VMEM((_K * _L,), jnp.float32),
            pltpu.VMEM((n_chunk,), jnp.float32),
            pltpu.VMEM((n_chunk,), jnp.float32),
            pltpu.VMEM((_L,), jnp.float32),
        ],
    )(w, rank, x)
    loss = jnp.sum(part) / n
    return out.reshape(shape), loss, loss
